# baseline (speedup 1.0000x reference)
import functools

import jax
import jax.numpy as jnp
from jax import lax
from jax.experimental import pallas as pl
from jax.experimental.pallas import tpu as pltpu
from jax.experimental.pallas import tpu_sc as plsc

N = 100000
D = 128
C = 40
NCHUNK = N // C
NW = 32
NT = -(-NCHUNK // NW)
NB = 5
NGROUP = -(-NT // NB)
CR = 160
NCOPY = N // CR

DO_COPY = True
DO_AVG = True


def _pool_body(inputs_hbm, idx_hbm, out_hbm, idx_all, gbuf, obuf, cbuf, *sems):
    sem_g = sems[0:NB]
    sem_s = sems[NB:2 * NB]
    sem_ci = sems[2 * NB:2 * NB + 2]
    sem_co = sems[2 * NB + 2:2 * NB + 4]
    wid = lax.axis_index("s") * 2 + lax.axis_index("c")
    sid = lax.axis_index("s")

    pltpu.sync_copy(idx_hbm.at[wid], idx_all)

    nt = jnp.minimum(NT, NCHUNK - wid * NT)

    def gather_start(t, b):
        pltpu.async_copy(inputs_hbm.at[idx_all.at[t]], gbuf.at[b], sem_g[b])

    def gather_wait(t, b):
        pltpu.make_async_copy(inputs_hbm.at[idx_all.at[t]], gbuf.at[b], sem_g[b]).wait()

    def store_start(t, b):
        base = (wid * NT + t) * C
        pltpu.async_copy(obuf.at[b], out_hbm.at[pl.ds(N + base, C)], sem_s[b])

    def store_wait(t, b):
        base = (wid * NT + t) * C
        pltpu.make_async_copy(obuf.at[b], out_hbm.at[pl.ds(N + base, C)], sem_s[b]).wait()

    def cin_copy(u, p, start):
        base = (wid + NW * u) * CR
        cp = pltpu.make_async_copy(inputs_hbm.at[pl.ds(base, CR)], cbuf.at[sid, p],
                                   sem_ci[p])
        cp.start() if start else cp.wait()

    def cout_copy(u, p, start):
        base = (wid + NW * u) * CR
        cp = pltpu.make_async_copy(cbuf.at[sid, p], out_hbm.at[pl.ds(base, CR)],
                                   sem_co[p])
        cp.start() if start else cp.wait()

    def copy_step(u, p):
        if not DO_COPY:
            return
        @pl.when(jnp.logical_and(u >= 2, wid + NW * (u - 2) < NCOPY))
        def _():
            cout_copy(u - 2, p, start=False)

        @pl.when(wid + NW * u < NCOPY)
        def _():
            cin_copy(u, p, start=True)

        @pl.when(jnp.logical_and(u >= 1, wid + NW * (u - 1) < NCOPY))
        def _():
            cin_copy(u - 1, 1 - p, start=False)
            cout_copy(u - 1, 1 - p, start=True)

    for b in range(NB):
        @pl.when(b < nt)
        def _(b=b):
            gather_start(b, b)

    def pair_body(h, carry):
        for gg in range(2):
            g = 2 * h + gg
            copy_step(2 * g, 0)
            copy_step(2 * g + 1, 1)
            for b in range(NB):
                @pl.when(g * NB + b < nt)
                def _(b=b, g=g):
                    t = g * NB + b
                    gather_wait(t, b)

                    @pl.when(t >= NB)
                    def _():
                        store_wait(t - NB, b)

                    def row_body(j, rc):
                        for r in range(4):
                            i = 4 * j + r
                            for q in range(D // 16):
                                av = gbuf[b, 2 * i, pl.ds(q * 16, 16)]
                                bv = gbuf[b, 2 * i + 1, pl.ds(q * 16, 16)]
                                obuf[b, i, pl.ds(q * 16, 16)] = (av + bv) * 0.5
                        return rc

                    if DO_AVG:
                        lax.fori_loop(0, C // 4, row_body, 0)
                    store_start(t, b)

                    @pl.when(t + NB < nt)
                    def _():
                        gather_start(t + NB, b)
        return carry

    lax.fori_loop(0, NGROUP // 2, pair_body, 0)

    if DO_COPY:
        for u, p in ((2 * NGROUP, 0), (2 * NGROUP + 1, 1)):
            @pl.when(jnp.logical_and(u >= 2, wid + NW * (u - 2) < NCOPY))
            def _(u=u, p=p):
                cout_copy(u - 2, p, start=False)

            @pl.when(jnp.logical_and(u >= 1, wid + NW * (u - 1) < NCOPY))
            def _(u=u, p=p):
                cin_copy(u - 1, 1 - p, start=False)
                cout_copy(u - 1, 1 - p, start=True)

    for b in range(NB):
        @pl.when(b < nt)
        def _(b=b):
            last_t = nt - 1 - lax.rem(nt - 1 - b, NB)
            store_wait(last_t, b)


@functools.partial(
    pl.kernel,
    mesh=plsc.VectorSubcoreMesh(core_axis_name="c", subcore_axis_name="s"),
    out_type=jax.ShapeDtypeStruct((2 * N, D), jnp.float32),
    scratch_types=[
        pltpu.VMEM((NT, 2 * C), jnp.int32),
        pltpu.VMEM((NB, 2 * C, D), jnp.float32),
        pltpu.VMEM((NB, C, D), jnp.float32),
        pltpu.VMEM_SHARED((16, 2, CR, D), jnp.float32),
    ] + [pltpu.SemaphoreType.DMA] * (2 * NB + 4),
)
def _pooled(inputs_hbm, idx_hbm, out_hbm, idx_all, gbuf, obuf, cbuf, *sems):
    _pool_body(inputs_hbm, idx_hbm, out_hbm, idx_all, gbuf, obuf, cbuf, *sems)


def kernel(inputs, pool_idx):
    flat = pool_idx.astype(jnp.int32).reshape(-1)
    flat = jnp.pad(flat, (0, NW * NT * 2 * C - 2 * N))
    idx3 = flat.reshape(NW, NT, 2 * C)
    return _pooled(inputs, idx3)


# speedup vs baseline: 1.0580x; 1.0580x over previous
import functools

import jax
import jax.numpy as jnp
from jax import lax
from jax.experimental import pallas as pl
from jax.experimental.pallas import tpu as pltpu
from jax.experimental.pallas import tpu_sc as plsc

N = 100000
D = 128
C = 40
NCHUNK = N // C
NW = 32
NT = -(-NCHUNK // NW)
NB = 4
NGROUP = -(-NT // NB)
CR = 200
NCOPY = N // CR


def _pool_body(inputs_hbm, half_hbm, idx_hbm, out_hbm, idx_all, obuf, cbuf, *sems):
    sem_g1 = sems[0:NB]
    sem_g2 = sems[NB:2 * NB]
    sem_s = sems[2 * NB:3 * NB]
    sem_ci = sems[3 * NB:3 * NB + 2]
    sem_co = sems[3 * NB + 2:3 * NB + 4]
    wid = lax.axis_index("s") * 2 + lax.axis_index("c")
    sid = lax.axis_index("s")

    pltpu.sync_copy(idx_hbm.at[wid], idx_all)

    nt = jnp.minimum(NT, NCHUNK - wid * NT)

    def g1(t, b, start):
        cp = pltpu.make_async_copy(half_hbm.at[idx_all.at[t, 0]], obuf.at[b], sem_g1[b])
        cp.start() if start else cp.wait()

    def g2(t, b, start):
        if start:
            pltpu.async_copy(half_hbm.at[idx_all.at[t, 1]], obuf.at[b],
                             sem_g2[b], add=True)
        else:
            pltpu.make_async_copy(half_hbm.at[idx_all.at[t, 1]], obuf.at[b],
                                  sem_g2[b]).wait()

    def st(t, b, start):
        base = (wid * NT + t) * C
        cp = pltpu.make_async_copy(obuf.at[b], out_hbm.at[pl.ds(N + base, C)], sem_s[b])
        cp.start() if start else cp.wait()

    def cin_copy(u, p, start):
        base = (wid + NW * u) * CR
        cp = pltpu.make_async_copy(inputs_hbm.at[pl.ds(base, CR)], cbuf.at[sid, p],
                                   sem_ci[p])
        cp.start() if start else cp.wait()

    def cout_copy(u, p, start):
        base = (wid + NW * u) * CR
        cp = pltpu.make_async_copy(cbuf.at[sid, p], out_hbm.at[pl.ds(base, CR)],
                                   sem_co[p])
        cp.start() if start else cp.wait()

    def copy_step(u, p):
        @pl.when(jnp.logical_and(u >= 2, wid + NW * (u - 2) < NCOPY))
        def _():
            cout_copy(u - 2, p, start=False)

        @pl.when(wid + NW * u < NCOPY)
        def _():
            cin_copy(u, p, start=True)

        @pl.when(jnp.logical_and(u >= 1, wid + NW * (u - 1) < NCOPY))
        def _():
            cin_copy(u - 1, 1 - p, start=False)
            cout_copy(u - 1, 1 - p, start=True)

    def pair_body(h, carry):
        for gg in range(2):
            g = 2 * h + gg
            copy_step(2 * g, 0)
            copy_step(2 * g + 1, 1)
            for b in range(NB):
                @pl.when(g * NB + b < nt)
                def _(b=b, g=g):
                    t = g * NB + b
                    g1(t, b, True)
                    g1(t, b, False)
                    g2(t, b, True)
                    g2(t, b, False)
                    st(t, b, True)
                    st(t, b, False)
        return carry

    lax.fori_loop(0, NGROUP // 2, pair_body, 0)

    for u, p in ((2 * NGROUP, 0), (2 * NGROUP + 1, 1)):
        @pl.when(jnp.logical_and(u >= 2, wid + NW * (u - 2) < NCOPY))
        def _(u=u, p=p):
            cout_copy(u - 2, p, start=False)

        @pl.when(jnp.logical_and(u >= 1, wid + NW * (u - 1) < NCOPY))
        def _(u=u, p=p):
            cin_copy(u - 1, 1 - p, start=False)
            cout_copy(u - 1, 1 - p, start=True)


@functools.partial(
    pl.kernel,
    mesh=plsc.VectorSubcoreMesh(core_axis_name="c", subcore_axis_name="s"),
    out_type=jax.ShapeDtypeStruct((2 * N, D), jnp.float32),
    scratch_types=[
        pltpu.VMEM((NT, 2, C), jnp.int32),
        pltpu.VMEM((NB, C, D), jnp.float32),
        pltpu.VMEM_SHARED((16, 2, CR, D), jnp.float32),
    ] + [pltpu.SemaphoreType.DMA] * (3 * NB + 4),
)
def _pooled(inputs_hbm, half_hbm, idx_hbm, out_hbm, idx_all, obuf, cbuf, *sems):
    _pool_body(inputs_hbm, half_hbm, idx_hbm, out_hbm, idx_all, obuf, cbuf, *sems)


def kernel(inputs, pool_idx):
    idx = pool_idx.astype(jnp.int32)
    idx = jnp.pad(idx, ((0, NW * NT * C - N), (0, 0)))
    idx4 = idx.reshape(NW, NT, C, 2).transpose(0, 1, 3, 2)
    return _pooled(inputs, inputs * 0.5, idx4)


# speedup vs baseline: 1.7872x; 1.6892x over previous
import functools

import jax
import jax.numpy as jnp
from jax import lax
from jax.experimental import pallas as pl
from jax.experimental.pallas import tpu as pltpu
from jax.experimental.pallas import tpu_sc as plsc

N = 100000
D = 128
C = 40
NCHUNK = N // C
NW = 32
NT = -(-NCHUNK // NW)
NB = 8
NGROUP = -(-NT // NB)
CR = 200
NCOPY = N // CR


def _pool_body(inputs_hbm, half_hbm, idx_hbm, out_hbm, idx_all, obuf, cbuf, *sems):
    sem_g1 = sems[0:NB]
    sem_g2 = sems[NB:2 * NB]
    sem_s = sems[2 * NB:3 * NB]
    sem_ci = sems[3 * NB:3 * NB + 2]
    sem_co = sems[3 * NB + 2:3 * NB + 4]
    wid = lax.axis_index("s") * 2 + lax.axis_index("c")
    sid = lax.axis_index("s")

    pltpu.sync_copy(idx_hbm.at[wid], idx_all)

    nt = jnp.minimum(NT, NCHUNK - wid * NT)

    def g1(t, b, start):
        cp = pltpu.make_async_copy(half_hbm.at[idx_all.at[t, 0]], obuf.at[b], sem_g1[b])
        cp.start() if start else cp.wait()

    def g2(t, b, start):
        if start:
            pltpu.async_copy(half_hbm.at[idx_all.at[t, 1]], obuf.at[b],
                             sem_g2[b], add=True)
        else:
            pltpu.make_async_copy(half_hbm.at[idx_all.at[t, 1]], obuf.at[b],
                                  sem_g2[b]).wait()

    def st(t, b, start):
        base = (wid * NT + t) * C
        cp = pltpu.make_async_copy(obuf.at[b], out_hbm.at[pl.ds(N + base, C)], sem_s[b])
        cp.start() if start else cp.wait()

    def cin_copy(u, p, start):
        base = (wid + NW * u) * CR
        cp = pltpu.make_async_copy(inputs_hbm.at[pl.ds(base, CR)], cbuf.at[sid, p],
                                   sem_ci[p])
        cp.start() if start else cp.wait()

    def cout_copy(u, p, start):
        base = (wid + NW * u) * CR
        cp = pltpu.make_async_copy(cbuf.at[sid, p], out_hbm.at[pl.ds(base, CR)],
                                   sem_co[p])
        cp.start() if start else cp.wait()

    def copy_step(u, p):
        @pl.when(jnp.logical_and(u >= 2, wid + NW * (u - 2) < NCOPY))
        def _():
            cout_copy(u - 2, p, start=False)

        @pl.when(wid + NW * u < NCOPY)
        def _():
            cin_copy(u, p, start=True)

        @pl.when(jnp.logical_and(u >= 1, wid + NW * (u - 1) < NCOPY))
        def _():
            cin_copy(u - 1, 1 - p, start=False)
            cout_copy(u - 1, 1 - p, start=True)

    for u in range(NB - 2):
        @pl.when(u < nt)
        def _(u=u):
            g1(u, u % NB, True)

    g1(0, 0, False)
    g2(0, 0, True)

    def pair_body(h, carry):
        for gg in range(2):
            g = 2 * h + gg
            copy_step(2 * g, 0)
            copy_step(2 * g + 1, 1)
            for b in range(NB):
                t = g * NB + b

                @pl.when(jnp.logical_and(t >= 2, t < nt))
                def _(t=t, b=b):
                    st(t - 2, (b - 2) % NB, False)

                @pl.when(t + NB - 2 < nt)
                def _(t=t, b=b):
                    g1(t + NB - 2, (b - 2) % NB, True)

                @pl.when(t + 1 < nt)
                def _(t=t, b=b):
                    g1(t + 1, (b + 1) % NB, False)
                    g2(t + 1, (b + 1) % NB, True)

                @pl.when(t < nt)
                def _(t=t, b=b):
                    g2(t, b, False)
                    st(t, b, True)
        return carry

    lax.fori_loop(0, NGROUP // 2, pair_body, 0)

    r2 = lax.rem(nt - 2, NB)
    r1 = lax.rem(nt - 1, NB)
    for b in range(NB):
        @pl.when(b == r2)
        def _(b=b):
            st(nt - 2, b, False)
    for b in range(NB):
        @pl.when(b == r1)
        def _(b=b):
            st(nt - 1, b, False)

    for u, p in ((2 * NGROUP, 0), (2 * NGROUP + 1, 1)):
        @pl.when(jnp.logical_and(u >= 2, wid + NW * (u - 2) < NCOPY))
        def _(u=u, p=p):
            cout_copy(u - 2, p, start=False)

        @pl.when(jnp.logical_and(u >= 1, wid + NW * (u - 1) < NCOPY))
        def _(u=u, p=p):
            cin_copy(u - 1, 1 - p, start=False)
            cout_copy(u - 1, 1 - p, start=True)


@functools.partial(
    pl.kernel,
    mesh=plsc.VectorSubcoreMesh(core_axis_name="c", subcore_axis_name="s"),
    out_type=jax.ShapeDtypeStruct((2 * N, D), jnp.float32),
    scratch_types=[
        pltpu.VMEM((NT, 2, C), jnp.int32),
        pltpu.VMEM((NB, C, D), jnp.float32),
        pltpu.VMEM_SHARED((16, 2, CR, D), jnp.float32),
    ] + [pltpu.SemaphoreType.DMA] * (3 * NB + 4),
)
def _pooled(inputs_hbm, half_hbm, idx_hbm, out_hbm, idx_all, obuf, cbuf, *sems):
    _pool_body(inputs_hbm, half_hbm, idx_hbm, out_hbm, idx_all, obuf, cbuf, *sems)


def kernel(inputs, pool_idx):
    idx = pool_idx.astype(jnp.int32)
    idx = jnp.pad(idx, ((0, NW * NT * C - N), (0, 0)))
    idx4 = idx.reshape(NW, NT, C, 2).transpose(0, 1, 3, 2)
    return _pooled(inputs, inputs * 0.5, idx4)
